# baseline (speedup 1.0000x reference)
import functools

import jax
import jax.numpy as jnp
from jax import lax
from jax.experimental import pallas as pl
from jax.experimental.pallas import tpu as pltpu
from jax.experimental.pallas import tpu_sc as plsc

D = 64
B = 16384
NF = 5
CH = 128
UV = 1000000
SV = 2 + 7 + 21 + 1000

_info = plsc.get_sparse_core_info()
NC = _info.num_cores
NS = _info.num_subcores
NW = NC * NS
BPW = B // NW
NCH = BPW // CH

_mesh = plsc.VectorSubcoreMesh(core_axis_name="c", subcore_axis_name="s")


@functools.partial(
    pl.kernel,
    out_type=jax.ShapeDtypeStruct((B, NF * D), jnp.float32),
    mesh=_mesh,
    compiler_params=pltpu.CompilerParams(use_tc_tiling_on_sc=False),
    scratch_types=[
        pltpu.VMEM((NF, NCH, CH), jnp.int32),
        pltpu.VMEM((NCH, CH), jnp.int32),
        pltpu.VMEM_SHARED((SV, D), jnp.float32),
        pltpu.VMEM((2, CH, 2 * D), jnp.float32),
        pltpu.VMEM((2, CH, D), jnp.float32),
        pltpu.VMEM((8, CH, D), jnp.float32),
        pltpu.SemaphoreType.DMA,
        pltpu.SemaphoreType.DMA,
        pltpu.SemaphoreType.DMA,
        pltpu.SemaphoreType.DMA,
        pltpu.SemaphoreType.DMA,
        pltpu.SemaphoreType.DMA,
        pltpu.SemaphoreType.DMA,
        pltpu.SemaphoreType.DMA,
        pltpu.SemaphoreType.DMA,
        pltpu.SemaphoreType.DMA,
    ],
)
def _emb_concat(idx_hbm, Wu2, Ws, out_hbm,
                idx_v, pidx_v, spm, pbuf, ubuf, sbuf,
                sg0, sg1, sw0, sw1, ssg0, ssg1, ssg2, ssg3, ssw0, ssw1):
    gsems = (sg0, sg1)
    wsems = (sw0, sw1)
    ssgsems = (ssg0, ssg1, ssg2, ssg3)
    sswsems = (ssw0, ssw1)

    sid = lax.axis_index("s")
    wid = sid * NC + lax.axis_index("c")

    @pl.when(sid == 0)
    def _():
        pltpu.sync_copy(Ws, spm)

    for f in range(NF):
        pltpu.sync_copy(idx_hbm.at[f, pl.ds(wid * NCH, NCH)], idx_v.at[f])
    for c in range(NCH):
        for g in range(CH // 16):
            u16 = idx_v[0, c, pl.ds(g * 16, 16)]
            pidx_v[c, pl.ds(g * 16, 16)] = u16 >> 1

    plsc.subcore_barrier()

    base = wid * BPW

    def ugather(c):
        return pltpu.async_copy(
            Wu2.at[pidx_v.at[c]], pbuf.at[c % 2], gsems[c % 2])

    def uwrite(c):
        return pltpu.async_copy(
            ubuf.at[c % 2],
            out_hbm.at[pl.ds(base + c * CH, CH), pl.ds(0, D)],
            wsems[c % 2])

    def uextract(c):
        pb = pbuf.at[c % 2]
        ub = ubuf.at[c % 2]

        def gbody(g, _):
            u16 = idx_v[0, c, pl.ds(g * 16, 16)]
            h16 = (u16 & 1) * D
            for l in range(16):
                b = g * 16 + l
                h = h16[l]
                for q in range(D // 16):
                    ub[b, pl.ds(q * 16, 16)] = pb[b, pl.ds(h + q * 16, 16)]
            return _

        lax.fori_loop(0, CH // 16, gbody, 0)

    ug = [None] * NCH
    uw = [None] * NCH
    sg = [None] * (NCH * 4)
    sw = [None] * (NCH * 4)

    ug[0] = ugather(0)
    for t in range(4):
        sg[t] = pltpu.async_copy(
            spm.at[idx_v.at[t + 1, 0]], sbuf.at[t], ssgsems[t])
    for c in range(NCH):
        if c + 1 < NCH:
            if c - 1 >= 0:
                uw[c - 1].wait()
            ug[c + 1] = ugather(c + 1)
        for t in range(4):
            k = c * 4 + t
            sg[k].wait()
            if k - 2 >= 0:
                sw[k - 2].wait()
            sw[k] = pltpu.async_copy(
                sbuf.at[(c % 2) * 4 + t],
                out_hbm.at[pl.ds(base + c * CH, CH), pl.ds((t + 1) * D, D)],
                sswsems[k % 2])
            if k + 4 < NCH * 4:
                c2, t2 = divmod(k + 4, 4)
                sg[k + 4] = pltpu.async_copy(
                    spm.at[idx_v.at[t2 + 1, c2]],
                    sbuf.at[(c2 % 2) * 4 + t2], ssgsems[t2])
        ug[c].wait()
        uextract(c)
        uw[c] = uwrite(c)
    uw[NCH - 2].wait()
    uw[NCH - 1].wait()
    sw[NCH * 4 - 2].wait()
    sw[NCH * 4 - 1].wait()


def kernel(user_id, gender, age, occupation, zip_code,
           W_user_id, W_gender, W_age, W_occupation, W_zip_code):
    idx = jnp.stack([user_id, gender, age + 2, occupation + 9,
                     zip_code + 30])
    idx = idx.reshape(NF, B // CH, CH)
    Ws = jnp.concatenate([W_gender, W_age, W_occupation, W_zip_code], axis=0)
    return _emb_concat(idx, W_user_id.reshape(UV // 2, 2 * D), Ws)
